# baseline (speedup 1.0000x reference)
import functools

import jax
import jax.numpy as jnp
from jax import lax
from jax.experimental import pallas as pl
from jax.experimental.pallas import tpu as pltpu
from jax.experimental.pallas import tpu_sc as plsc

_B = 16384
_F = 128
_LANES = 16
_NC = 2
_NS = 16
_NW = _NC * _NS
_BPW = _B // _NW
_CHUNK = 64
_NCHUNK = _BPW // _CHUNK
_NPAIR = _NCHUNK // 2
_MW = _F // 4


def _sc_body(vtab_hbm, gtab_hbm, tgt_hbm, vals_hbm,
             bmv_hbm, bmg_hbm, msk_hbm,
             idx_v, vrow, grow, val, msk, in_sem, out_sem):
    wid = lax.axis_index("s") * _NC + lax.axis_index("c")
    base = wid * _BPW

    pltpu.sync_copy(tgt_hbm.at[pl.ds(base, _BPW)], idx_v)

    def read_descs(b, c):
        idx_sl = idx_v.at[pl.ds(c * _CHUNK, _CHUNK)]
        row0 = base + c * _CHUNK
        return (
            pltpu.make_async_copy(vtab_hbm.at[idx_sl], vrow[b], in_sem),
            pltpu.make_async_copy(gtab_hbm.at[idx_sl], grow[b], in_sem),
            pltpu.make_async_copy(vals_hbm.at[pl.ds(row0, _CHUNK)], val[b],
                                  in_sem),
        )

    def vg_write_descs(b, c):
        row0 = base + c * _CHUNK
        return (
            pltpu.make_async_copy(vrow[b], bmv_hbm.at[pl.ds(row0, _CHUNK)],
                                  out_sem),
            pltpu.make_async_copy(grow[b], bmg_hbm.at[pl.ds(row0, _CHUNK)],
                                  out_sem),
        )

    def m_write_desc(b, c):
        row0 = base + c * _CHUNK
        return pltpu.make_async_copy(msk[b], msk_hbm.at[pl.ds(row0, _CHUNK)],
                                     out_sem)

    def compute(b):
        dnums = lax.GatherDimensionNumbers(
            offset_dims=(), collapsed_slice_dims=(0,), start_index_map=(0,))

        def perm(v, idx):
            return lax.gather(v, idx[:, None], dnums, (1,),
                              mode=lax.GatherScatterMode.PROMISE_IN_BOUNDS)

        lanes = lax.iota(jnp.int32, _LANES)
        zero = jnp.full((_LANES,), 0, jnp.int32)
        one = jnp.full((_LANES,), 1, jnp.int32)
        roll_idx = [jnp.minimum(lanes + k, _LANES - 1) for k in (1, 2, 3)]
        z_idx = [jnp.clip(4 * (lanes - 4 * q), 0, _LANES - 1)
                 for q in range(4)]
        q_sel = [(lanes >= 4 * q) & (lanes < 4 * q + 4) for q in range(1, 4)]

        def ew(i, _):
            for blk in range(2):
                zs = []
                for q in range(4):
                    sl = pl.ds(blk * 64 + q * _LANES, _LANES)
                    mv = vrow[b][i, sl]
                    mg = grow[b][i, sl]
                    va = val[b][i, sl]
                    m = (mv < 0.0) & (va > 0.0) & (mg < 0.0)
                    w = jnp.where(m, one, zero)
                    y = (w
                         | (perm(w, roll_idx[0]) << 8)
                         | (perm(w, roll_idx[1]) << 16)
                         | (perm(w, roll_idx[2]) << 24))
                    zs.append(perm(y, z_idx[q]))
                w = zs[0]
                for q in range(1, 4):
                    w = jnp.where(q_sel[q - 1], zs[q], w)
                msk[b][i, pl.ds(blk * _LANES, _LANES)] = w
            return _

        lax.fori_loop(0, _CHUNK, ew, None)

    def stage(b, c):
        for d in read_descs(b, c):
            d.wait()
        for d in vg_write_descs(b, c):
            d.start()
        compute(b)
        m_write_desc(b, c).start()

    def recycle(b, c, k):
        for d in vg_write_descs(b, c):
            d.wait()
        m_write_desc(b, c).wait()

        @pl.when(k < _NPAIR - 1)
        def _():
            for d in read_descs(b, c + 2):
                d.start()

    for d in read_descs(0, 0):
        d.start()
    for d in read_descs(1, 1):
        d.start()

    def body(k, carry):
        c0 = 2 * k
        c1 = c0 + 1
        stage(0, c0)
        stage(1, c1)
        recycle(0, c0, k)
        recycle(1, c1, k)
        return carry

    lax.fori_loop(0, _NPAIR, body, 0)


@jax.jit
def _run(values, targets, value_memory, grad_memory):
    mesh = plsc.VectorSubcoreMesh(
        core_axis_name="c", subcore_axis_name="s",
        num_cores=_NC, num_subcores=_NS)
    f = functools.partial(
        pl.kernel,
        out_type=[
            jax.ShapeDtypeStruct((_B, _F), jnp.float32),
            jax.ShapeDtypeStruct((_B, _F), jnp.float32),
            jax.ShapeDtypeStruct((_B, _MW), jnp.int32),
        ],
        mesh=mesh,
        scratch_types=[
            pltpu.VMEM((_BPW,), jnp.int32),
            [pltpu.VMEM((_CHUNK, _F), jnp.float32) for _ in range(2)],
            [pltpu.VMEM((_CHUNK, _F), jnp.float32) for _ in range(2)],
            [pltpu.VMEM((_CHUNK, _F), jnp.float32) for _ in range(2)],
            [pltpu.VMEM((_CHUNK, _MW), jnp.int32) for _ in range(2)],
            pltpu.SemaphoreType.DMA,
            pltpu.SemaphoreType.DMA,
        ],
    )(_sc_body)
    return f(value_memory, grad_memory, targets, values)


def kernel(values, targets, value_memory, grad_memory):
    bmv, bmg, mw = _run(values, targets.astype(jnp.int32),
                        value_memory, grad_memory)
    mb = lax.bitcast_convert_type(mw, jnp.int8)
    return bmv, bmg, mb.reshape(_B, _F).astype(jnp.bool_)


# speedup vs baseline: 1.7524x; 1.7524x over previous
import functools

import jax
import jax.numpy as jnp
from jax import lax
from jax.experimental import pallas as pl
from jax.experimental.pallas import tpu as pltpu
from jax.experimental.pallas import tpu_sc as plsc

_B = 16384
_F = 128
_LANES = 16
_NC = 2
_NS = 16
_NW = _NC * _NS
_BPW = _B // _NW
_CHUNK = 64
_NCHUNK = _BPW // _CHUNK
_NBANK = 3


def _sc_body(vtab_hbm, gtab_hbm, tgt_hbm, vals_hbm,
             bmv_hbm, bmg_hbm, msk_hbm,
             idx_v, vrow, grow, val, msk, in_sem, out_sem):
    wid = lax.axis_index("s") * _NC + lax.axis_index("c")
    base = wid * _BPW

    pltpu.sync_copy(tgt_hbm.at[pl.ds(base, _BPW)], idx_v)

    def read_descs(c):
        b = c % _NBANK
        idx_sl = idx_v.at[pl.ds(c * _CHUNK, _CHUNK)]
        row0 = base + c * _CHUNK
        return (
            pltpu.make_async_copy(vtab_hbm.at[idx_sl], vrow[b], in_sem),
            pltpu.make_async_copy(gtab_hbm.at[idx_sl], grow[b], in_sem),
            pltpu.make_async_copy(vals_hbm.at[pl.ds(row0, _CHUNK)], val[b],
                                  in_sem),
        )

    def write_descs(c):
        b = c % _NBANK
        row0 = base + c * _CHUNK
        return (
            pltpu.make_async_copy(vrow[b], bmv_hbm.at[pl.ds(row0, _CHUNK)],
                                  out_sem),
            pltpu.make_async_copy(grow[b], bmg_hbm.at[pl.ds(row0, _CHUNK)],
                                  out_sem),
            pltpu.make_async_copy(msk[b], msk_hbm.at[pl.ds(row0, _CHUNK)],
                                  out_sem),
        )

    def compute(c):
        b = c % _NBANK

        def ew(i, _):
            for j in range(_F // _LANES):
                sl = pl.ds(j * _LANES, _LANES)
                mv = vrow[b][i, sl]
                mg = grow[b][i, sl]
                va = val[b][i, sl]
                m = (mv < 0.0) & (va > 0.0) & (mg < 0.0)
                msk[b][i, sl] = jnp.where(
                    m, jnp.full((_LANES,), 1, jnp.int32),
                    jnp.full((_LANES,), 0, jnp.int32))
            return _

        lax.fori_loop(0, _CHUNK, ew, None)

    for d in read_descs(0):
        d.start()
    for d in read_descs(1):
        d.start()
    for c in range(_NCHUNK):
        vgw, ggw, mgw = write_descs(c)
        for d in read_descs(c):
            d.wait()
        vgw.start()
        ggw.start()
        compute(c)
        mgw.start()
        if c + 2 < _NCHUNK:
            if c >= 1:
                for d in write_descs(c - 1):
                    d.wait()
            for d in read_descs(c + 2):
                d.start()
    for c in (_NCHUNK - 3, _NCHUNK - 2, _NCHUNK - 1):
        for d in write_descs(c):
            d.wait()


@jax.jit
def _run(values, targets, value_memory, grad_memory):
    mesh = plsc.VectorSubcoreMesh(
        core_axis_name="c", subcore_axis_name="s",
        num_cores=_NC, num_subcores=_NS)
    f = functools.partial(
        pl.kernel,
        out_type=[
            jax.ShapeDtypeStruct((_B, _F), jnp.float32),
            jax.ShapeDtypeStruct((_B, _F), jnp.float32),
            jax.ShapeDtypeStruct((_B, _F), jnp.int32),
        ],
        mesh=mesh,
        scratch_types=[
            pltpu.VMEM((_BPW,), jnp.int32),
            [pltpu.VMEM((_CHUNK, _F), jnp.float32) for _ in range(_NBANK)],
            [pltpu.VMEM((_CHUNK, _F), jnp.float32) for _ in range(_NBANK)],
            [pltpu.VMEM((_CHUNK, _F), jnp.float32) for _ in range(_NBANK)],
            [pltpu.VMEM((_CHUNK, _F), jnp.int32) for _ in range(_NBANK)],
            pltpu.SemaphoreType.DMA,
            pltpu.SemaphoreType.DMA,
        ],
    )(_sc_body)
    return f(value_memory, grad_memory, targets, values)


def kernel(values, targets, value_memory, grad_memory):
    bmv, bmg, msk = _run(values, targets.astype(jnp.int32),
                         value_memory, grad_memory)
    return bmv, bmg, msk.astype(jnp.bool_)


# speedup vs baseline: 1.7679x; 1.0089x over previous
import functools

import jax
import jax.numpy as jnp
from jax import lax
from jax.experimental import pallas as pl
from jax.experimental.pallas import tpu as pltpu
from jax.experimental.pallas import tpu_sc as plsc

_B = 16384
_F = 128
_LANES = 16
_NC = 2
_NS = 16
_NW = _NC * _NS
_BPW = _B // _NW
_CHUNK = 64
_NCHUNK = _BPW // _CHUNK
_NPAIR = _NCHUNK // 2


def _sc_body(vtab_hbm, gtab_hbm, tgt_hbm, vals_hbm,
             bmv_hbm, bmg_hbm, msk_hbm,
             idx_v, vrow, grow, val, msk, in_sem, out_sem):
    wid = lax.axis_index("s") * _NC + lax.axis_index("c")
    base = wid * _BPW

    pltpu.sync_copy(tgt_hbm.at[pl.ds(base, _BPW)], idx_v)

    def read_descs(b, c):
        idx_sl = idx_v.at[pl.ds(c * _CHUNK, _CHUNK)]
        row0 = base + c * _CHUNK
        return (
            pltpu.make_async_copy(vtab_hbm.at[idx_sl], vrow[b], in_sem),
            pltpu.make_async_copy(gtab_hbm.at[idx_sl], grow[b], in_sem),
            pltpu.make_async_copy(vals_hbm.at[pl.ds(row0, _CHUNK)], val[b],
                                  in_sem),
        )

    def vg_write_descs(b, c):
        row0 = base + c * _CHUNK
        return (
            pltpu.make_async_copy(vrow[b], bmv_hbm.at[pl.ds(row0, _CHUNK)],
                                  out_sem),
            pltpu.make_async_copy(grow[b], bmg_hbm.at[pl.ds(row0, _CHUNK)],
                                  out_sem),
        )

    def m_write_desc(b, c):
        row0 = base + c * _CHUNK
        return pltpu.make_async_copy(msk[b], msk_hbm.at[pl.ds(row0, _CHUNK)],
                                     out_sem)

    def compute(b):
        def ew(i, _):
            for j in range(_F // _LANES):
                sl = pl.ds(j * _LANES, _LANES)
                mv = vrow[b][i, sl]
                mg = grow[b][i, sl]
                va = val[b][i, sl]
                m = (mv < 0.0) & (va > 0.0) & (mg < 0.0)
                msk[b][i, sl] = jnp.where(
                    m, jnp.full((_LANES,), 1, jnp.int32),
                    jnp.full((_LANES,), 0, jnp.int32))
            return _

        lax.fori_loop(0, _CHUNK, ew, None)

    def stage(b, c):
        for d in read_descs(b, c):
            d.wait()
        for d in vg_write_descs(b, c):
            d.start()
        compute(b)
        m_write_desc(b, c).start()

    def recycle(b, c, k):
        for d in vg_write_descs(b, c):
            d.wait()
        m_write_desc(b, c).wait()

        @pl.when(k < _NPAIR - 1)
        def _():
            for d in read_descs(b, c + 2):
                d.start()

    for d in read_descs(0, 0):
        d.start()
    for d in read_descs(1, 1):
        d.start()

    def body(k, carry):
        c0 = 2 * k
        c1 = c0 + 1
        stage(0, c0)
        stage(1, c1)
        recycle(0, c0, k)
        recycle(1, c1, k)
        return carry

    lax.fori_loop(0, _NPAIR, body, 0)


@jax.jit
def _run(values, targets, value_memory, grad_memory):
    mesh = plsc.VectorSubcoreMesh(
        core_axis_name="c", subcore_axis_name="s",
        num_cores=_NC, num_subcores=_NS)
    f = functools.partial(
        pl.kernel,
        out_type=[
            jax.ShapeDtypeStruct((_B, _F), jnp.float32),
            jax.ShapeDtypeStruct((_B, _F), jnp.float32),
            jax.ShapeDtypeStruct((_B, _F), jnp.int32),
        ],
        mesh=mesh,
        scratch_types=[
            pltpu.VMEM((_BPW,), jnp.int32),
            [pltpu.VMEM((_CHUNK, _F), jnp.float32) for _ in range(2)],
            [pltpu.VMEM((_CHUNK, _F), jnp.float32) for _ in range(2)],
            [pltpu.VMEM((_CHUNK, _F), jnp.float32) for _ in range(2)],
            [pltpu.VMEM((_CHUNK, _F), jnp.int32) for _ in range(2)],
            pltpu.SemaphoreType.DMA,
            pltpu.SemaphoreType.DMA,
        ],
    )(_sc_body)
    return f(value_memory, grad_memory, targets, values)


def kernel(values, targets, value_memory, grad_memory):
    bmv, bmg, msk = _run(values, targets.astype(jnp.int32),
                         value_memory, grad_memory)
    return bmv, bmg, msk.astype(jnp.bool_)
